# baseline (speedup 1.0000x reference)
import jax
import jax.numpy as jnp
from jax.experimental import pallas as pl
from jax.experimental.pallas import tpu as pltpu

N = 10000
IN_DIM = 128
H1 = 128
H2 = 64

ENC_R = 400
ENC_WAYS = 10
DEC_R = 80


def _enc_kernel(*refs):
    g_refs = refs[:ENC_WAYS]
    (f_hbm, w1_ref, lns_ref, lnb_ref, w2_ref, b2_ref, w3_ref, b3_ref,
     eps_ref, mu_ref, logvar_ref, z_ref, f_vmem, sup_ref, sem) = refs[ENC_WAYS:]
    i = pl.program_id(0)

    @pl.when(i == 0)
    def _():
        cp = pltpu.make_async_copy(f_hbm, f_vmem, sem)
        cp.start()
        cp.wait()
        sup_ref[...] = jnp.dot(f_vmem[...], w1_ref[...],
                               preferred_element_type=jnp.float32
                               ).astype(jnp.bfloat16)

    h1 = jnp.concatenate(
        [jnp.dot(r[...].astype(jnp.bfloat16), sup_ref[...],
                 preferred_element_type=jnp.float32) for r in g_refs],
        axis=0)
    h1 = jnp.maximum(h1, 0.0)
    mean = jnp.mean(h1, axis=-1, keepdims=True)
    var = jnp.mean((h1 - mean) ** 2, axis=-1, keepdims=True)
    h = (h1 - mean) / jnp.sqrt(var + 1e-5) * lns_ref[...] + lnb_ref[...]
    mu = jnp.dot(h, w2_ref[...], preferred_element_type=jnp.float32) + b2_ref[...]
    logvar = jnp.dot(h, w3_ref[...], preferred_element_type=jnp.float32) + b3_ref[...]
    z = eps_ref[...] * jnp.exp(logvar) + mu
    mu_ref[...] = mu
    logvar_ref[...] = logvar
    z_ref[...] = z


def _dec_kernel(zi_ref, zt_hbm, adj_ref, zt_vmem, sem):
    i = pl.program_id(0)

    @pl.when(i == 0)
    def _():
        cp = pltpu.make_async_copy(zt_hbm, zt_vmem, sem)
        cp.start()
        cp.wait()

    adj_ref[...] = jnp.dot(zi_ref[...], zt_vmem[...],
                           preferred_element_type=jnp.float32)


def _g_spec(k):
    sub = ENC_R // ENC_WAYS
    return pl.BlockSpec((sub, N), lambda i, k=k: (ENC_WAYS * i + k, 0))


@jax.jit
def kernel(g, features, W1, ln_scale, ln_bias, W2, b2, W3, b3):
    eps = jax.random.normal(jax.random.key(42), (N, H2), dtype=jnp.float32)
    lns = ln_scale.reshape(1, H1)
    lnb = ln_bias.reshape(1, H1)
    b2r = b2.reshape(1, H2)
    b3r = b3.reshape(1, H2)

    mu, logvar, z = pl.pallas_call(
        _enc_kernel,
        grid=(N // ENC_R,),
        in_specs=[_g_spec(k) for k in range(ENC_WAYS)] + [
            pl.BlockSpec(memory_space=pltpu.MemorySpace.HBM),
            pl.BlockSpec((IN_DIM, H1), lambda i: (0, 0)),
            pl.BlockSpec((1, H1), lambda i: (0, 0)),
            pl.BlockSpec((1, H1), lambda i: (0, 0)),
            pl.BlockSpec((H1, H2), lambda i: (0, 0)),
            pl.BlockSpec((1, H2), lambda i: (0, 0)),
            pl.BlockSpec((H1, H2), lambda i: (0, 0)),
            pl.BlockSpec((1, H2), lambda i: (0, 0)),
            pl.BlockSpec((ENC_R, H2), lambda i: (i, 0)),
        ],
        out_specs=[
            pl.BlockSpec((ENC_R, H2), lambda i: (i, 0)),
            pl.BlockSpec((ENC_R, H2), lambda i: (i, 0)),
            pl.BlockSpec((ENC_R, H2), lambda i: (i, 0)),
        ],
        out_shape=[
            jax.ShapeDtypeStruct((N, H2), jnp.float32),
            jax.ShapeDtypeStruct((N, H2), jnp.float32),
            jax.ShapeDtypeStruct((N, H2), jnp.float32),
        ],
        scratch_shapes=[
            pltpu.VMEM((N, IN_DIM), jnp.float32),
            pltpu.VMEM((N, H1), jnp.bfloat16),
            pltpu.SemaphoreType.DMA,
        ],
    )(*([g] * ENC_WAYS), features, W1, lns, lnb, W2, b2r, W3, b3r, eps)

    zb = z.astype(jnp.bfloat16)
    ztb = zb.T

    adj = pl.pallas_call(
        _dec_kernel,
        grid=(N // DEC_R,),
        in_specs=[
            pl.BlockSpec((DEC_R, H2), lambda i: (i, 0)),
            pl.BlockSpec(memory_space=pltpu.MemorySpace.HBM),
        ],
        out_specs=pl.BlockSpec((DEC_R, N), lambda i: (i, 0)),
        out_shape=jax.ShapeDtypeStruct((N, N), jnp.float32),
        scratch_shapes=[
            pltpu.VMEM((H2, N), jnp.bfloat16),
            pltpu.SemaphoreType.DMA,
        ],
    )(zb, ztb)

    return (adj, mu, logvar, z)


# speedup vs baseline: 1.0891x; 1.0891x over previous
import jax
import jax.numpy as jnp
from jax.experimental import pallas as pl
from jax.experimental.pallas import tpu as pltpu

N = 10000
IN_DIM = 128
H1 = 128
H2 = 64

ENC_R = 400
NSUB = 10
SUB = ENC_R // NSUB
DEC_R = 400


def _enc_kernel(g_hbm, f_hbm, w1_ref, lns_ref, lnb_ref, w2_ref, b2_ref,
                w3_ref, b3_ref, eps_ref, mu_ref, logvar_ref, z_ref,
                gbuf, f_vmem, sup_ref, fsem, sems):
    i = pl.program_id(0)
    nsteps = pl.num_programs(0)

    def g_copy(step, slot, j):
        return pltpu.make_async_copy(
            g_hbm.at[pl.ds(step * ENC_R + j * SUB, SUB), :],
            gbuf.at[slot, pl.ds(j * SUB, SUB), :],
            sems.at[slot, j])

    slot = jax.lax.rem(i, 2)

    @pl.when(i == 0)
    def _():
        for j in range(NSUB):
            g_copy(0, 0, j).start()
        cp = pltpu.make_async_copy(f_hbm, f_vmem, fsem)
        cp.start()
        cp.wait()
        sup_ref[...] = jnp.dot(f_vmem[...], w1_ref[...],
                               preferred_element_type=jnp.float32
                               ).astype(jnp.bfloat16)

    @pl.when(i + 1 < nsteps)
    def _():
        for j in range(NSUB):
            g_copy(i + 1, 1 - slot, j).start()

    for j in range(NSUB):
        g_copy(i, slot, j).wait()

    h1 = jnp.concatenate(
        [jnp.dot(gbuf[slot, j * SUB:(j + 1) * SUB, :].astype(jnp.bfloat16),
                 sup_ref[...], preferred_element_type=jnp.float32)
         for j in range(NSUB)],
        axis=0)
    h1 = jnp.maximum(h1, 0.0)
    mean = jnp.mean(h1, axis=-1, keepdims=True)
    var = jnp.mean((h1 - mean) ** 2, axis=-1, keepdims=True)
    h = (h1 - mean) / jnp.sqrt(var + 1e-5) * lns_ref[...] + lnb_ref[...]
    mu = jnp.dot(h, w2_ref[...], preferred_element_type=jnp.float32) + b2_ref[...]
    logvar = jnp.dot(h, w3_ref[...], preferred_element_type=jnp.float32) + b3_ref[...]
    z = eps_ref[...] * jnp.exp(logvar) + mu
    mu_ref[...] = mu
    logvar_ref[...] = logvar
    z_ref[...] = z


def _dec_kernel(zi_ref, zt_hbm, adj_ref, zt_vmem, sem):
    i = pl.program_id(0)

    @pl.when(i == 0)
    def _():
        cp = pltpu.make_async_copy(zt_hbm, zt_vmem, sem)
        cp.start()
        cp.wait()

    adj_ref[...] = jnp.dot(zi_ref[...], zt_vmem[...],
                           preferred_element_type=jnp.float32)


@jax.jit
def kernel(g, features, W1, ln_scale, ln_bias, W2, b2, W3, b3):
    eps = jax.random.normal(jax.random.key(42), (N, H2), dtype=jnp.float32)
    lns = ln_scale.reshape(1, H1)
    lnb = ln_bias.reshape(1, H1)
    b2r = b2.reshape(1, H2)
    b3r = b3.reshape(1, H2)

    mu, logvar, z = pl.pallas_call(
        _enc_kernel,
        grid=(N // ENC_R,),
        in_specs=[
            pl.BlockSpec(memory_space=pltpu.MemorySpace.HBM),
            pl.BlockSpec(memory_space=pltpu.MemorySpace.HBM),
            pl.BlockSpec((IN_DIM, H1), lambda i: (0, 0)),
            pl.BlockSpec((1, H1), lambda i: (0, 0)),
            pl.BlockSpec((1, H1), lambda i: (0, 0)),
            pl.BlockSpec((H1, H2), lambda i: (0, 0)),
            pl.BlockSpec((1, H2), lambda i: (0, 0)),
            pl.BlockSpec((H1, H2), lambda i: (0, 0)),
            pl.BlockSpec((1, H2), lambda i: (0, 0)),
            pl.BlockSpec((ENC_R, H2), lambda i: (i, 0)),
        ],
        out_specs=[
            pl.BlockSpec((ENC_R, H2), lambda i: (i, 0)),
            pl.BlockSpec((ENC_R, H2), lambda i: (i, 0)),
            pl.BlockSpec((ENC_R, H2), lambda i: (i, 0)),
        ],
        out_shape=[
            jax.ShapeDtypeStruct((N, H2), jnp.float32),
            jax.ShapeDtypeStruct((N, H2), jnp.float32),
            jax.ShapeDtypeStruct((N, H2), jnp.float32),
        ],
        scratch_shapes=[
            pltpu.VMEM((2, ENC_R, N), jnp.float32),
            pltpu.VMEM((N, IN_DIM), jnp.float32),
            pltpu.VMEM((N, H1), jnp.bfloat16),
            pltpu.SemaphoreType.DMA,
            pltpu.SemaphoreType.DMA((2, NSUB)),
        ],
    )(g, features, W1, lns, lnb, W2, b2r, W3, b3r, eps)

    zb = z.astype(jnp.bfloat16)
    ztb = zb.T

    adj = pl.pallas_call(
        _dec_kernel,
        grid=(N // DEC_R,),
        in_specs=[
            pl.BlockSpec((DEC_R, H2), lambda i: (i, 0)),
            pl.BlockSpec(memory_space=pltpu.MemorySpace.HBM),
        ],
        out_specs=pl.BlockSpec((DEC_R, N), lambda i: (i, 0)),
        out_shape=jax.ShapeDtypeStruct((N, N), jnp.float32),
        scratch_shapes=[
            pltpu.VMEM((H2, N), jnp.bfloat16),
            pltpu.SemaphoreType.DMA,
        ],
    )(zb, ztb)

    return (adj, mu, logvar, z)
